# Initial kernel scaffold; baseline (speedup 1.0000x reference)
import jax
import jax.numpy as jnp
from jax.experimental import pallas as pl


def kernel(h, edge_index, W, a):
    raise NotImplementedError("write your pallas kernel here")


# speedup vs baseline: 9.2082x; 9.2082x over previous
import functools

import jax
import jax.numpy as jnp
from jax import lax
from jax.experimental import pallas as pl
from jax.experimental.pallas import tpu as pltpu
from jax.experimental.pallas import tpu_sc as plsc

ALPHA = 0.2
LANES = 16
CP = 48
NC = 2
NS = 16
NW = NC * NS
CH = 128


def _project_body(h_ref, w_ref, a2_ref, hw_ref, s_ref):
    hW = jnp.dot(h_ref[...], w_ref[...], preferred_element_type=jnp.float32)
    b = hW.shape[0]
    ones = jnp.ones((b, 1), jnp.float32)
    zeros = jnp.zeros((b, CP - hW.shape[1] - 1), jnp.float32)
    hw_ref[...] = jnp.concatenate([hW, ones, zeros], axis=1)
    s_ref[...] = lax.dot_general(
        hW, a2_ref[...], (((1,), (1,)), ((), ())),
        preferred_element_type=jnp.float32)


def _project(h, W, a2):
    n, d = h.shape
    c = W.shape[1]
    blk = 2000
    grid = n // blk
    return pl.pallas_call(
        _project_body,
        grid=(grid,),
        in_specs=[
            pl.BlockSpec((blk, d), lambda i: (i, 0)),
            pl.BlockSpec((d, c), lambda i: (0, 0)),
            pl.BlockSpec((2, c), lambda i: (0, 0)),
        ],
        out_specs=[
            pl.BlockSpec((blk, CP), lambda i: (i, 0)),
            pl.BlockSpec((blk, 2), lambda i: (i, 0)),
        ],
        out_shape=[
            jax.ShapeDtypeStruct((n, CP), jnp.float32),
            jax.ShapeDtypeStruct((n, 2), jnp.float32),
        ],
    )(h, W, a2)


def _sc_edge(hw_pad, s12, src2d, dst2d, zeros, nch):
    n = hw_pad.shape[0]
    nb = src2d.shape[0] // NW
    rpt = 640
    last_rows = n - (NS - 1) * rpt

    @functools.partial(
        pl.kernel,
        out_type=jax.ShapeDtypeStruct((NC, n, CP), jnp.float32),
        mesh=plsc.VectorSubcoreMesh(core_axis_name="c", subcore_axis_name="s"),
        compiler_params=pltpu.CompilerParams(needs_layout_passes=False, use_tc_tiling_on_sc=False),
        scratch_types=[
            pltpu.VMEM((2 * n,), jnp.float32),
            pltpu.VMEM((nb, CH), jnp.int32),
            pltpu.VMEM((nb, CH), jnp.int32),
            pltpu.VMEM((CH,), jnp.float32),
            pltpu.VMEM((CH, CP), jnp.float32),
            pltpu.VMEM_SHARED((n, CP), jnp.float32),
            pltpu.SemaphoreType.DMA,
        ],
    )
    def k(hw_hbm, s12_hbm, src_hbm, dst_hbm, z_hbm, out_hbm,
          s12_v, src_v, dst_v, w_v, rows_v, acc_s, sem):
        sid = lax.axis_index("s")
        cid = lax.axis_index("c")
        wid = sid * NC + cid

        @pl.when(sid < NS - 1)
        def _():
            pltpu.sync_copy(z_hbm, acc_s.at[pl.ds(sid * rpt, rpt)])

        @pl.when(sid == NS - 1)
        def _():
            pltpu.sync_copy(z_hbm.at[pl.ds(0, last_rows)],
                            acc_s.at[pl.ds((NS - 1) * rpt, last_rows)])

        pltpu.sync_copy(s12_hbm, s12_v)
        pltpu.sync_copy(src_hbm.at[pl.ds(wid * nb, nb)], src_v)
        pltpu.sync_copy(dst_hbm.at[pl.ds(wid * nb, nb)], dst_v)

        plsc.subcore_barrier()


        def chunk_body(j, carry):
            cp = pltpu.async_copy(hw_hbm.at[dst_v.at[j]], rows_v, sem)
            for g in range(CH // LANES):
                srcv = src_v[j, pl.ds(g * LANES, LANES)]
                dstv = dst_v[j, pl.ds(g * LANES, LANES)]
                s1 = plsc.load_gather(s12_v, [srcv * 2])
                s2 = plsc.load_gather(s12_v, [dstv * 2 + 1])
                logit = s1 + s2
                lk = jnp.where(logit >= 0.0, logit, logit * ALPHA)
                w_v[pl.ds(g * LANES, LANES)] = jnp.exp(-lk)
            cp.wait()

            def scale_body(i, c2):
                wi = plsc.load_gather(w_v, [jnp.full((LANES,), i, jnp.int32)])
                for cc in range(CP // LANES):
                    sl = pl.ds(cc * LANES, LANES)
                    rows_v[i, sl] = rows_v[i, sl] * wi
                return c2

            lax.fori_loop(0, CH, scale_body, 0)
            pltpu.sync_copy(rows_v, acc_s.at[src_v.at[j]], add=True)
            return carry

        n_k = jnp.clip(nch - nb * wid, 0, nb)
        lax.fori_loop(0, n_k, chunk_body, 0)

        plsc.subcore_barrier()

        @pl.when(sid < NS - 1)
        def _():
            pltpu.sync_copy(acc_s.at[pl.ds(sid * rpt, rpt)],
                            out_hbm.at[cid, pl.ds(sid * rpt, rpt)])

        @pl.when(sid == NS - 1)
        def _():
            pltpu.sync_copy(
                acc_s.at[pl.ds((NS - 1) * rpt, last_rows)],
                out_hbm.at[cid, pl.ds((NS - 1) * rpt, last_rows)])

    return k(hw_pad, s12, src2d, dst2d, zeros)


def _finalize_body(acc_ref, out_ref):
    a0 = acc_ref[0]
    a1 = acc_ref[1]
    c = out_ref.shape[1]
    num = a0[:, :c] + a1[:, :c]
    den = a0[:, c:c + 1] + a1[:, c:c + 1]
    hp = num / den
    out_ref[...] = jnp.where(hp > 0.0, hp, jnp.exp(hp) - 1.0)


def _finalize(accum, c):
    n = accum.shape[1]
    blk = 1000
    grid = n // blk
    return pl.pallas_call(
        _finalize_body,
        grid=(grid,),
        in_specs=[pl.BlockSpec((NC, blk, CP), lambda i: (0, i, 0))],
        out_specs=pl.BlockSpec((blk, c), lambda i: (i, 0)),
        out_shape=jax.ShapeDtypeStruct((n, c), jnp.float32),
    )(accum)


def kernel(h, edge_index, W, a):
    n = h.shape[0]
    c = W.shape[1]
    e = edge_index.shape[1]
    hw_pad, s12 = _project(h, W, a.reshape(2, c))
    nch = e // CH
    nb = -(-nch // NW)
    pad = nb * NW - nch
    src2d = jnp.pad(edge_index[0], (0, pad * CH)).reshape(nb * NW, CH)
    dst2d = jnp.pad(edge_index[1], (0, pad * CH)).reshape(nb * NW, CH)
    zeros = jnp.zeros((640, CP), jnp.float32)
    accum = _sc_edge(hw_pad, s12.reshape(2 * n), src2d, dst2d, zeros, nch)
    return _finalize(accum, c)
